# Initial kernel scaffold; baseline (speedup 1.0000x reference)
import jax
import jax.numpy as jnp
from jax.experimental import pallas as pl


def kernel(xi, h_A, h_B, E_val, I_val, L_val, q_val, W1, b1, W2, b2, Ww, bw, Wm, bm):
    raise NotImplementedError("write your pallas kernel here")


# speedup vs baseline: 1.5119x; 1.5119x over previous
import jax
import jax.numpy as jnp
from jax.experimental import pallas as pl
from jax.experimental.pallas import tpu as pltpu

_BS = 4000


def _mlp_kernel(sc_ref, hA_ref, hB_ref, W1s_ref, W1a_ref, W1b_ref, b1_ref,
                W2_ref, b2_ref, Wh_ref, bh_ref, out_ref):
    f = jnp.dot(hA_ref[...], W1a_ref[...], preferred_element_type=jnp.float32)
    f = f + jnp.dot(hB_ref[...], W1b_ref[...], preferred_element_type=jnp.float32)
    f = f + jnp.dot(sc_ref[...], W1s_ref[...], preferred_element_type=jnp.float32)
    f = jnp.tanh(f + b1_ref[...])
    f = jnp.tanh(jnp.dot(f, W2_ref[...], preferred_element_type=jnp.float32)
                 + b2_ref[...])
    out_ref[...] = (jnp.dot(f, Wh_ref[...], preferred_element_type=jnp.float32)
                    + bh_ref[...])


def kernel(xi, h_A, h_B, E_val, I_val, L_val, q_val,
           W1, b1, W2, b2, Ww, bw, Wm, bm):
    B, H = h_A.shape
    D1 = W1.shape[1]
    D2 = W2.shape[1]

    zeros = jnp.zeros((B, 3), dtype=xi.dtype)
    scal = jnp.concatenate([xi, E_val, I_val, L_val, q_val, zeros], axis=-1)
    W1s = jnp.concatenate(
        [W1[0:1], W1[1 + 2 * H:], jnp.zeros((3, D1), W1.dtype)], axis=0)
    W1a = W1[1:1 + H]
    W1b = W1[1 + H:1 + 2 * H]
    Wh = jnp.concatenate([Ww, Wm], axis=1)
    bh = jnp.concatenate([bw, bm]).reshape(1, 2)

    grid = (B // _BS,)
    row = lambda i: (i, 0)
    rep = lambda i: (0, 0)

    out = pl.pallas_call(
        _mlp_kernel,
        grid=grid,
        in_specs=[
            pl.BlockSpec((_BS, 8), row),
            pl.BlockSpec((_BS, H), row),
            pl.BlockSpec((_BS, H), row),
            pl.BlockSpec((8, D1), rep),
            pl.BlockSpec((H, D1), rep),
            pl.BlockSpec((H, D1), rep),
            pl.BlockSpec((1, D1), rep),
            pl.BlockSpec((D1, D2), rep),
            pl.BlockSpec((1, D2), rep),
            pl.BlockSpec((D2, 2), rep),
            pl.BlockSpec((1, 2), rep),
        ],
        out_specs=pl.BlockSpec((_BS, 2), row),
        out_shape=jax.ShapeDtypeStruct((B, 2), jnp.float32),
        compiler_params=pltpu.CompilerParams(
            dimension_semantics=("arbitrary",)),
    )(scal, h_A, h_B, W1s, W1a, W1b, b1.reshape(1, D1),
      W2, b2.reshape(1, D2), Wh, bh)

    return (out[:, 0:1], out[:, 1:2])


# speedup vs baseline: 1.5720x; 1.0397x over previous
import jax
import jax.numpy as jnp
from jax.experimental import pallas as pl
from jax.experimental.pallas import tpu as pltpu

_BS = 8000


def _mlp_kernel(sc_ref, hA_ref, hB_ref, W1s_ref, W1a_ref, W1b_ref, b1_ref,
                W2_ref, b2_ref, Wh_ref, bh_ref, out_ref):
    f = jnp.dot(hA_ref[...], W1a_ref[...], preferred_element_type=jnp.float32)
    f = f + jnp.dot(hB_ref[...], W1b_ref[...], preferred_element_type=jnp.float32)
    f = f + jnp.dot(sc_ref[...], W1s_ref[...], preferred_element_type=jnp.float32)
    f = jnp.tanh(f + b1_ref[...])
    f = jnp.tanh(jnp.dot(f, W2_ref[...], preferred_element_type=jnp.float32)
                 + b2_ref[...])
    out_ref[...] = (jnp.dot(f, Wh_ref[...], preferred_element_type=jnp.float32)
                    + bh_ref[...])


def kernel(xi, h_A, h_B, E_val, I_val, L_val, q_val,
           W1, b1, W2, b2, Ww, bw, Wm, bm):
    B, H = h_A.shape
    D1 = W1.shape[1]
    D2 = W2.shape[1]

    zeros = jnp.zeros((B, 3), dtype=xi.dtype)
    scal = jnp.concatenate([xi, E_val, I_val, L_val, q_val, zeros], axis=-1)
    W1s = jnp.concatenate(
        [W1[0:1], W1[1 + 2 * H:], jnp.zeros((3, D1), W1.dtype)], axis=0)
    W1a = W1[1:1 + H]
    W1b = W1[1 + H:1 + 2 * H]
    Wh = jnp.concatenate([Ww, Wm], axis=1)
    bh = jnp.concatenate([bw, bm]).reshape(1, 2)

    grid = (B // _BS,)
    row = lambda i: (i, 0)
    rep = lambda i: (0, 0)

    out = pl.pallas_call(
        _mlp_kernel,
        grid=grid,
        in_specs=[
            pl.BlockSpec((_BS, 8), row),
            pl.BlockSpec((_BS, H), row),
            pl.BlockSpec((_BS, H), row),
            pl.BlockSpec((8, D1), rep),
            pl.BlockSpec((H, D1), rep),
            pl.BlockSpec((H, D1), rep),
            pl.BlockSpec((1, D1), rep),
            pl.BlockSpec((D1, D2), rep),
            pl.BlockSpec((1, D2), rep),
            pl.BlockSpec((D2, 2), rep),
            pl.BlockSpec((1, 2), rep),
        ],
        out_specs=pl.BlockSpec((_BS, 2), row),
        out_shape=jax.ShapeDtypeStruct((B, 2), jnp.float32),
        compiler_params=pltpu.CompilerParams(
            dimension_semantics=("arbitrary",)),
    )(scal, h_A, h_B, W1s, W1a, W1b, b1.reshape(1, D1),
      W2, b2.reshape(1, D2), Wh, bh)

    return (out[:, 0:1], out[:, 1:2])
